# Initial kernel scaffold; baseline (speedup 1.0000x reference)
import functools

import jax
import jax.numpy as jnp
from jax import lax
from jax.experimental import pallas as pl
from jax.experimental.pallas import tpu as pltpu
from jax.experimental.pallas import tpu_sc as plsc

N_NODES = 100000
N_EDGES = 6400000
EPS = 1e-15

NC = 2
NS = 16
NW = NC * NS
E_PER_W = N_EDGES // NW
B = 4000
NCHUNK = E_PER_W // B


def _rsqrt(s):
    bits = plsc.bitcast(s, jnp.int32)
    r = plsc.bitcast(jnp.int32(0x5F3759DF) - (bits >> 1), jnp.float32)
    for _ in range(3):
        r = r * (1.5 - 0.5 * s * r * r)
    return r


def _distance_body(rx, ry, rz, idx_i_hbm, idx_j_hbm, off_hbm, out_hbm,
                   ii_v, ij_v, xi_v, yi_v, zi_v, xj_v, yj_v, zj_v,
                   off_v, out_v, sem):
    wid = lax.axis_index("s") * NC + lax.axis_index("c")
    base = wid * E_PER_W
    lanes = lax.iota(jnp.int32, 16)
    c0 = jnp.zeros((16,), jnp.int32)
    c1 = jnp.ones((16,), jnp.int32)
    c2 = jnp.full((16,), 2, jnp.int32)

    def chunk_body(c, carry):
        start = base + c * B
        pltpu.sync_copy(idx_i_hbm.at[pl.ds(start, B)], ii_v)
        pltpu.sync_copy(idx_j_hbm.at[pl.ds(start, B)], ij_v)
        g1 = pltpu.async_copy(rx.at[ii_v], xi_v, sem)
        g2 = pltpu.async_copy(ry.at[ii_v], yi_v, sem)
        g3 = pltpu.async_copy(rz.at[ii_v], zi_v, sem)
        g4 = pltpu.async_copy(rx.at[ij_v], xj_v, sem)
        g5 = pltpu.async_copy(ry.at[ij_v], yj_v, sem)
        g6 = pltpu.async_copy(rz.at[ij_v], zj_v, sem)
        pltpu.sync_copy(off_hbm.at[pl.ds(start, B)], off_v)
        g1.wait()
        g2.wait()
        g3.wait()
        g4.wait()
        g5.wait()
        g6.wait()

        def vec_body(k, carry2):
            s = pl.ds(k * 16, 16)
            rows = lanes + k * 16
            ox = plsc.load_gather(off_v, [rows, c0])
            oy = plsc.load_gather(off_v, [rows, c1])
            oz = plsc.load_gather(off_v, [rows, c2])
            dx = xi_v[s] - xj_v[s] - ox + EPS
            dy = yi_v[s] - yj_v[s] - oy + EPS
            dz = zi_v[s] - zj_v[s] - oz + EPS
            ss = dx * dx + dy * dy + dz * dz
            out_v[s] = ss * _rsqrt(ss)
            return carry2

        lax.fori_loop(0, B // 16, vec_body, 0, unroll=2)
        pltpu.sync_copy(out_v, out_hbm.at[pl.ds(start, B)])
        return carry

    lax.fori_loop(0, NCHUNK, chunk_body, 0)


@functools.partial(
    pl.kernel,
    out_type=jax.ShapeDtypeStruct((N_EDGES,), jnp.float32),
    mesh=plsc.VectorSubcoreMesh(core_axis_name="c", subcore_axis_name="s"),
    scratch_types=[
        pltpu.VMEM((B,), jnp.int32),
        pltpu.VMEM((B,), jnp.int32),
        pltpu.VMEM((B,), jnp.float32),
        pltpu.VMEM((B,), jnp.float32),
        pltpu.VMEM((B,), jnp.float32),
        pltpu.VMEM((B,), jnp.float32),
        pltpu.VMEM((B,), jnp.float32),
        pltpu.VMEM((B,), jnp.float32),
        pltpu.VMEM((B, 3), jnp.float32),
        pltpu.VMEM((B,), jnp.float32),
        pltpu.SemaphoreType.DMA,
    ],
)
def _distance_kernel(rx, ry, rz, idx_i_hbm, idx_j_hbm, off_hbm, out_hbm,
                     ii_v, ij_v, xi_v, yi_v, zi_v, xj_v, yj_v, zj_v,
                     off_v, out_v, sem):
    _distance_body(rx, ry, rz, idx_i_hbm, idx_j_hbm, off_hbm, out_hbm,
                   ii_v, ij_v, xi_v, yi_v, zi_v, xj_v, yj_v, zj_v,
                   off_v, out_v, sem)


def kernel(Ra, idx_i, idx_j, offsets):
    rx = jnp.ascontiguousarray(Ra[:, 0])
    ry = jnp.ascontiguousarray(Ra[:, 1])
    rz = jnp.ascontiguousarray(Ra[:, 2])
    return _distance_kernel(rx, ry, rz, idx_i, idx_j, offsets)


# speedup vs baseline: 4.7008x; 4.7008x over previous
import functools

import jax
import jax.numpy as jnp
from jax import lax
from jax.experimental import pallas as pl
from jax.experimental.pallas import tpu as pltpu
from jax.experimental.pallas import tpu_sc as plsc

N_NODES = 100000
N_EDGES = 6400000
EPS = 1e-15

NC = 2
NS = 16
NW = NC * NS
E_PER_W = N_EDGES // NW
B = 4000
NCHUNK = E_PER_W // B


def _rsqrt(s):
    bits = plsc.bitcast(s, jnp.int32)
    r = plsc.bitcast(jnp.int32(0x5F3759DF) - (bits >> 1), jnp.float32)
    for _ in range(3):
        r = r * (1.5 - 0.5 * s * r * r)
    return r


def _distance_body(rx, ry, rz, idx_i_hbm, idx_j_hbm, off_hbm, out_hbm,
                   ii_v, ij_v, xi_v, yi_v, zi_v, xj_v, yj_v, zj_v,
                   off_v, out_v, sem):
    wid = lax.axis_index("s") * NC + lax.axis_index("c")
    base = wid * E_PER_W
    lanes3 = lax.iota(jnp.int32, 16) * 3

    def chunk_body(c, carry):
        start = base + c * B
        pltpu.sync_copy(idx_i_hbm.at[pl.ds(start, B)], ii_v)
        pltpu.sync_copy(idx_j_hbm.at[pl.ds(start, B)], ij_v)
        g1 = pltpu.async_copy(rx.at[ii_v], xi_v, sem)
        g2 = pltpu.async_copy(ry.at[ii_v], yi_v, sem)
        g3 = pltpu.async_copy(rz.at[ii_v], zi_v, sem)
        g4 = pltpu.async_copy(rx.at[ij_v], xj_v, sem)
        g5 = pltpu.async_copy(ry.at[ij_v], yj_v, sem)
        g6 = pltpu.async_copy(rz.at[ij_v], zj_v, sem)
        pltpu.sync_copy(off_hbm.at[pl.ds(start * 3, B * 3)], off_v)
        g1.wait()
        g2.wait()
        g3.wait()
        g4.wait()
        g5.wait()
        g6.wait()

        def vec_body(k, carry2):
            s = pl.ds(k * 16, 16)
            fbase = lanes3 + k * 48
            ox = plsc.load_gather(off_v, [fbase])
            oy = plsc.load_gather(off_v, [fbase + 1])
            oz = plsc.load_gather(off_v, [fbase + 2])
            dx = xi_v[s] - xj_v[s] - ox + EPS
            dy = yi_v[s] - yj_v[s] - oy + EPS
            dz = zi_v[s] - zj_v[s] - oz + EPS
            ss = dx * dx + dy * dy + dz * dz
            out_v[s] = ss * _rsqrt(ss)
            return carry2

        lax.fori_loop(0, B // 16, vec_body, 0, unroll=2)
        pltpu.sync_copy(out_v, out_hbm.at[pl.ds(start, B)])
        return carry

    lax.fori_loop(0, NCHUNK, chunk_body, 0)


@functools.partial(
    pl.kernel,
    out_type=jax.ShapeDtypeStruct((N_EDGES,), jnp.float32),
    mesh=plsc.VectorSubcoreMesh(core_axis_name="c", subcore_axis_name="s"),
    compiler_params=pltpu.CompilerParams(needs_layout_passes=False),
    scratch_types=[
        pltpu.VMEM((B,), jnp.int32),
        pltpu.VMEM((B,), jnp.int32),
        pltpu.VMEM((B,), jnp.float32),
        pltpu.VMEM((B,), jnp.float32),
        pltpu.VMEM((B,), jnp.float32),
        pltpu.VMEM((B,), jnp.float32),
        pltpu.VMEM((B,), jnp.float32),
        pltpu.VMEM((B,), jnp.float32),
        pltpu.VMEM((B * 3,), jnp.float32),
        pltpu.VMEM((B,), jnp.float32),
        pltpu.SemaphoreType.DMA,
    ],
)
def _distance_kernel(rx, ry, rz, idx_i_hbm, idx_j_hbm, off_hbm, out_hbm,
                     ii_v, ij_v, xi_v, yi_v, zi_v, xj_v, yj_v, zj_v,
                     off_v, out_v, sem):
    _distance_body(rx, ry, rz, idx_i_hbm, idx_j_hbm, off_hbm, out_hbm,
                   ii_v, ij_v, xi_v, yi_v, zi_v, xj_v, yj_v, zj_v,
                   off_v, out_v, sem)


def kernel(Ra, idx_i, idx_j, offsets):
    rx = Ra[:, 0]
    ry = Ra[:, 1]
    rz = Ra[:, 2]
    return _distance_kernel(rx, ry, rz, idx_i, idx_j, offsets.reshape(-1))
